# baseline (speedup 1.0000x reference)
import functools

import jax
import jax.numpy as jnp
from jax import lax
from jax.experimental import pallas as pl
from jax.experimental.pallas import tpu as pltpu
from jax.experimental.pallas import tpu_sc as plsc

BATCH = 16384
EMBED = 128

_NC, _NS = 2, 16
_NW = _NC * _NS
_B_PER_W = BATCH // _NW


def _make_gather():
    mesh = plsc.VectorSubcoreMesh(core_axis_name="c", subcore_axis_name="s")

    @functools.partial(
        pl.kernel,
        mesh=mesh,
        out_type=[
            jax.ShapeDtypeStruct((BATCH, EMBED), jnp.float32),
            jax.ShapeDtypeStruct((BATCH, EMBED), jnp.float32),
        ],
        scratch_types=[
            pltpu.VMEM((_B_PER_W,), jnp.int32),
            pltpu.VMEM((_B_PER_W, EMBED), jnp.float32),
            pltpu.SemaphoreType.DMA,
        ],
    )
    def gather_k(users_hbm, movies_hbm, ut_hbm, mt_hbm, u_out, m_out,
                 idx_v, rows_v, sem):
        wid = lax.axis_index("s") * _NC + lax.axis_index("c")
        base = wid * _B_PER_W
        pltpu.sync_copy(users_hbm.at[pl.ds(base, _B_PER_W)], idx_v)
        pltpu.async_copy(ut_hbm.at[idx_v], rows_v, sem).wait()
        pltpu.sync_copy(rows_v, u_out.at[pl.ds(base, _B_PER_W)])
        pltpu.sync_copy(movies_hbm.at[pl.ds(base, _B_PER_W)], idx_v)
        pltpu.async_copy(mt_hbm.at[idx_v], rows_v, sem).wait()
        pltpu.sync_copy(rows_v, m_out.at[pl.ds(base, _B_PER_W)])

    return gather_k


_gather = _make_gather()

_BLK = 8192


def _mlp_body(u_ref, m_ref, w1a_ref, w1b_ref, b1_ref, w2_ref, b2_ref,
              w3t_ref, b3_ref, o_ref):
    h = jnp.dot(u_ref[...], w1a_ref[...], preferred_element_type=jnp.float32)
    h += jnp.dot(m_ref[...], w1b_ref[...], preferred_element_type=jnp.float32)
    h = jnp.maximum(h + b1_ref[...], 0.0)
    h = jnp.maximum(
        jnp.dot(h, w2_ref[...], preferred_element_type=jnp.float32)
        + b2_ref[...], 0.0)
    o_ref[...] = lax.dot_general(
        w3t_ref[...], h, (((1,), (1,)), ((), ())),
        preferred_element_type=jnp.float32) + b3_ref[0, 0]


def _mlp(u, m, W1, b1, W2, b2, W3, b3):
    w1a, w1b = W1[:EMBED], W1[EMBED:]
    grid = BATCH // _BLK
    out_row = pl.pallas_call(
        _mlp_body,
        grid=(grid,),
        in_specs=[
            pl.BlockSpec((_BLK, EMBED), lambda i: (i, 0)),
            pl.BlockSpec((_BLK, EMBED), lambda i: (i, 0)),
            pl.BlockSpec((EMBED, 128), lambda i: (0, 0)),
            pl.BlockSpec((EMBED, 128), lambda i: (0, 0)),
            pl.BlockSpec((1, 128), lambda i: (0, 0)),
            pl.BlockSpec((128, 64), lambda i: (0, 0)),
            pl.BlockSpec((1, 64), lambda i: (0, 0)),
            pl.BlockSpec((1, 64), lambda i: (0, 0)),
            pl.BlockSpec((1, 1), lambda i: (0, 0)),
        ],
        out_specs=pl.BlockSpec((1, _BLK), lambda i: (0, i)),
        out_shape=jax.ShapeDtypeStruct((1, BATCH), jnp.float32),
    )(u, m, w1a, w1b, b1.reshape(1, 128), W2, b2.reshape(1, 64),
      W3.reshape(1, 64), b3.reshape(1, 1))
    return out_row.reshape(BATCH, 1)


def kernel(users, movies, user_table, movie_table, W1, b1, W2, b2, W3, b3):
    u, m = _gather(users.astype(jnp.int32), movies.astype(jnp.int32),
                   user_table, movie_table)
    return _mlp(u, m, W1, b1, W2, b2, W3, b3)


# speedup vs baseline: 1.0058x; 1.0058x over previous
import functools

import jax
import jax.numpy as jnp
from jax import lax
from jax.experimental import pallas as pl
from jax.experimental.pallas import tpu as pltpu
from jax.experimental.pallas import tpu_sc as plsc

BATCH = 16384
EMBED = 128

_NC, _NS = 2, 16
_NW = _NC * _NS
_B_PER_W = BATCH // _NW


def _make_gather():
    mesh = plsc.VectorSubcoreMesh(core_axis_name="c", subcore_axis_name="s")

    chunk = _B_PER_W // 2

    @functools.partial(
        pl.kernel,
        mesh=mesh,
        out_type=[
            jax.ShapeDtypeStruct((BATCH, EMBED), jnp.float32),
            jax.ShapeDtypeStruct((BATCH, EMBED), jnp.float32),
        ],
        scratch_types=[
            pltpu.VMEM((chunk,), jnp.int32),
            pltpu.VMEM((chunk,), jnp.int32),
            pltpu.VMEM((chunk,), jnp.int32),
            pltpu.VMEM((chunk,), jnp.int32),
            pltpu.VMEM((chunk, EMBED), jnp.float32),
            pltpu.VMEM((chunk, EMBED), jnp.float32),
            pltpu.VMEM((chunk, EMBED), jnp.float32),
            pltpu.SemaphoreType.DMA,
            pltpu.SemaphoreType.DMA,
            pltpu.SemaphoreType.DMA,
            pltpu.SemaphoreType.DMA,
            pltpu.SemaphoreType.DMA,
            pltpu.SemaphoreType.DMA,
            pltpu.SemaphoreType.DMA,
        ],
    )
    def gather_k(users_hbm, movies_hbm, ut_hbm, mt_hbm, u_out, m_out,
                 iu0, iu1, im0, im1, bufa, bufb, bufc,
                 sga, sgb, sgc, soa, sob, soc, sod):
        wid = lax.axis_index("s") * _NC + lax.axis_index("c")
        base = wid * _B_PER_W
        pltpu.sync_copy(users_hbm.at[pl.ds(base, chunk)], iu0)
        pltpu.sync_copy(users_hbm.at[pl.ds(base + chunk, chunk)], iu1)
        ga = pltpu.async_copy(ut_hbm.at[iu0], bufa, sga)
        gb = pltpu.async_copy(ut_hbm.at[iu1], bufb, sgb)
        pltpu.sync_copy(movies_hbm.at[pl.ds(base, chunk)], im0)
        pltpu.sync_copy(movies_hbm.at[pl.ds(base + chunk, chunk)], im1)
        gc = pltpu.async_copy(mt_hbm.at[im0], bufc, sgc)
        ga.wait()
        oa = pltpu.async_copy(bufa, u_out.at[pl.ds(base, chunk)], soa)
        gb.wait()
        ob = pltpu.async_copy(bufb, u_out.at[pl.ds(base + chunk, chunk)], sob)
        oa.wait()
        gd = pltpu.async_copy(mt_hbm.at[im1], bufa, sga)
        gc.wait()
        oc = pltpu.async_copy(bufc, m_out.at[pl.ds(base, chunk)], soc)
        gd.wait()
        od = pltpu.async_copy(bufa, m_out.at[pl.ds(base + chunk, chunk)], sod)
        ob.wait()
        oc.wait()
        od.wait()

    return gather_k


_gather = _make_gather()

_BLK = 4096


def _mlp_body(u_ref, m_ref, w1a_ref, w1b_ref, b1_ref, w2_ref, b2_ref,
              w3t_ref, b3_ref, o_ref):
    h = jnp.dot(u_ref[...], w1a_ref[...], preferred_element_type=jnp.float32)
    h += jnp.dot(m_ref[...], w1b_ref[...], preferred_element_type=jnp.float32)
    h = jnp.maximum(h + b1_ref[...], 0.0)
    h = jnp.maximum(
        jnp.dot(h, w2_ref[...], preferred_element_type=jnp.float32)
        + b2_ref[...], 0.0)
    o_ref[...] = lax.dot_general(
        w3t_ref[...], h, (((1,), (1,)), ((), ())),
        preferred_element_type=jnp.float32) + b3_ref[0, 0]


def _mlp(u, m, W1, b1, W2, b2, W3, b3):
    w1a, w1b = W1[:EMBED], W1[EMBED:]
    grid = BATCH // _BLK
    out_row = pl.pallas_call(
        _mlp_body,
        grid=(grid,),
        in_specs=[
            pl.BlockSpec((_BLK, EMBED), lambda i: (i, 0)),
            pl.BlockSpec((_BLK, EMBED), lambda i: (i, 0)),
            pl.BlockSpec((EMBED, 128), lambda i: (0, 0)),
            pl.BlockSpec((EMBED, 128), lambda i: (0, 0)),
            pl.BlockSpec((1, 128), lambda i: (0, 0)),
            pl.BlockSpec((128, 64), lambda i: (0, 0)),
            pl.BlockSpec((1, 64), lambda i: (0, 0)),
            pl.BlockSpec((1, 64), lambda i: (0, 0)),
            pl.BlockSpec((1, 1), lambda i: (0, 0)),
        ],
        out_specs=pl.BlockSpec((1, _BLK), lambda i: (0, i)),
        out_shape=jax.ShapeDtypeStruct((1, BATCH), jnp.float32),
    )(u, m, w1a, w1b, b1.reshape(1, 128), W2, b2.reshape(1, 64),
      W3.reshape(1, 64), b3.reshape(1, 1))
    return out_row.reshape(BATCH, 1)


def kernel(users, movies, user_table, movie_table, W1, b1, W2, b2, W3, b3):
    u, m = _gather(users.astype(jnp.int32), movies.astype(jnp.int32),
                   user_table, movie_table)
    return _mlp(u, m, W1, b1, W2, b2, W3, b3)


# speedup vs baseline: 1.0120x; 1.0062x over previous
import functools

import jax
import jax.numpy as jnp
from jax import lax
from jax.experimental import pallas as pl
from jax.experimental.pallas import tpu as pltpu
from jax.experimental.pallas import tpu_sc as plsc

BATCH = 16384
EMBED = 128

_NC, _NS = 2, 16
_NW = _NC * _NS
_B_PER_W = BATCH // _NW


def _make_gather():
    mesh = plsc.VectorSubcoreMesh(core_axis_name="c", subcore_axis_name="s")

    @functools.partial(
        pl.kernel,
        mesh=mesh,
        out_type=[
            jax.ShapeDtypeStruct((BATCH, EMBED), jnp.float32),
            jax.ShapeDtypeStruct((BATCH, EMBED), jnp.float32),
        ],
        scratch_types=[
            pltpu.VMEM((_B_PER_W,), jnp.int32),
            pltpu.VMEM((_B_PER_W, EMBED), jnp.float32),
            pltpu.SemaphoreType.DMA,
        ],
    )
    def gather_k(users_hbm, movies_hbm, ut_hbm, mt_hbm, u_out, m_out,
                 idx_v, rows_v, sem):
        wid = lax.axis_index("s") * _NC + lax.axis_index("c")
        base = wid * _B_PER_W
        pltpu.sync_copy(users_hbm.at[pl.ds(base, _B_PER_W)], idx_v)
        pltpu.async_copy(ut_hbm.at[idx_v], rows_v, sem).wait()
        pltpu.sync_copy(rows_v, u_out.at[pl.ds(base, _B_PER_W)])
        pltpu.sync_copy(movies_hbm.at[pl.ds(base, _B_PER_W)], idx_v)
        pltpu.async_copy(mt_hbm.at[idx_v], rows_v, sem).wait()
        pltpu.sync_copy(rows_v, m_out.at[pl.ds(base, _B_PER_W)])

    return gather_k


_gather = _make_gather()

_BLK = 4096


def _mlp_body(u_ref, m_ref, w1a_ref, w1b_ref, b1_ref, w2_ref, b2_ref,
              w3t_ref, b3_ref, o_ref):
    h = jnp.dot(u_ref[...], w1a_ref[...], preferred_element_type=jnp.float32)
    h += jnp.dot(m_ref[...], w1b_ref[...], preferred_element_type=jnp.float32)
    h = jnp.maximum(h + b1_ref[...], 0.0)
    h = jnp.maximum(
        jnp.dot(h, w2_ref[...], preferred_element_type=jnp.float32)
        + b2_ref[...], 0.0)
    o_ref[...] = lax.dot_general(
        w3t_ref[...], h, (((1,), (1,)), ((), ())),
        preferred_element_type=jnp.float32) + b3_ref[0, 0]


def _mlp(u, m, W1, b1, W2, b2, W3, b3):
    w1a, w1b = W1[:EMBED], W1[EMBED:]
    grid = BATCH // _BLK
    out_row = pl.pallas_call(
        _mlp_body,
        grid=(grid,),
        in_specs=[
            pl.BlockSpec((_BLK, EMBED), lambda i: (i, 0)),
            pl.BlockSpec((_BLK, EMBED), lambda i: (i, 0)),
            pl.BlockSpec((EMBED, 128), lambda i: (0, 0)),
            pl.BlockSpec((EMBED, 128), lambda i: (0, 0)),
            pl.BlockSpec((1, 128), lambda i: (0, 0)),
            pl.BlockSpec((128, 64), lambda i: (0, 0)),
            pl.BlockSpec((1, 64), lambda i: (0, 0)),
            pl.BlockSpec((1, 64), lambda i: (0, 0)),
            pl.BlockSpec((1, 1), lambda i: (0, 0)),
        ],
        out_specs=pl.BlockSpec((1, _BLK), lambda i: (0, i)),
        out_shape=jax.ShapeDtypeStruct((1, BATCH), jnp.float32),
    )(u, m, w1a, w1b, b1.reshape(1, 128), W2, b2.reshape(1, 64),
      W3.reshape(1, 64), b3.reshape(1, 1))
    return out_row.reshape(BATCH, 1)


def kernel(users, movies, user_table, movie_table, W1, b1, W2, b2, W3, b3):
    u, m = _gather(users.astype(jnp.int32), movies.astype(jnp.int32),
                   user_table, movie_table)
    return _mlp(u, m, W1, b1, W2, b2, W3, b3)
